# baseline (speedup 1.0000x reference)
import dataclasses

import jax
import jax.numpy as jnp
from jax import lax
from jax.experimental import pallas as pl
from jax.experimental.pallas import tpu as pltpu
from jax.experimental.pallas import tpu_sc as plsc

N = 10000
E = 320000
F = 128
H = 128
NP = 10240
NC = 2
NS = 16
NW = NC * NS
CHUNK = 128
NCH = 80
EPT = NCH * CHUNK
EP = NW * EPT
NR = NP // NS
RB = 1024

HALF = NP // NC
ACC2 = 5376
ZB2 = ACC2 // NS
MY = HALF // NS

_mesh = plsc.VectorSubcoreMesh(core_axis_name="c", subcore_axis_name="s")

_sc_params = pltpu.CompilerParams()
if "needs_layout_passes" in pltpu.CompilerParams.__dataclass_fields__:
    _sc_params = dataclasses.replace(_sc_params, needs_layout_passes=False)


def _deg_kernel(dst_hbm, out_hbm, didx_v, ones_v, zb_v, acc_sh, sem):
    c = lax.axis_index("c")
    s = lax.axis_index("s")
    w = c * NS + s

    pltpu.async_copy(dst_hbm.at[w], didx_v, sem)

    @pl.loop(0, CHUNK, step=16)
    def _(i):
        ones_v[pl.ds(i, 16)] = jnp.full((16,), 1.0, jnp.float32)

    @pl.loop(0, NR, step=16)
    def _(i):
        zb_v[pl.ds(i, 16)] = jnp.zeros((16,), jnp.float32)

    pltpu.sync_copy(zb_v, acc_sh.at[pl.ds(s * NR, NR)])
    pltpu.make_async_copy(dst_hbm.at[w], didx_v, sem).wait()
    plsc.subcore_barrier()

    @pl.loop(0, NCH)
    def _(k):
        pltpu.async_copy(ones_v, acc_sh.at[didx_v.at[k]], sem, add=True)

    @pl.loop(0, NCH)
    def _(k):
        pltpu.make_async_copy(ones_v, acc_sh.at[didx_v.at[0]], sem).wait()

    plsc.subcore_barrier()
    pltpu.sync_copy(acc_sh.at[pl.ds(s * NR, NR)],
                    out_hbm.at[c, pl.ds(s * NR, NR)])


def _sc_degree(dst3):
    return pl.kernel(
        _deg_kernel,
        out_type=jax.ShapeDtypeStruct((NC, NP), jnp.float32),
        mesh=_mesh,
        compiler_params=_sc_params,
        scratch_types=[
            pltpu.VMEM((NCH, CHUNK), jnp.int32),
            pltpu.VMEM((CHUNK,), jnp.float32),
            pltpu.VMEM((NR,), jnp.float32),
            pltpu.VMEM_SHARED((NP,), jnp.float32),
            pltpu.SemaphoreType.DMA,
        ],
    )(dst3)


def _agg_kernel(src_hbm, dst_hbm, y_hbm, out_hbm, sidx_v, didx0_v, didx1_v,
                rows0_v, rows1_v, acc_sh, semA, semB, semD0, semD1):
    c = lax.axis_index("c")
    s = lax.axis_index("s")
    w = c * NS + s

    pltpu.async_copy(src_hbm.at[w], sidx_v, semD0)

    @pl.loop(0, CHUNK)
    def _(r):
        @pl.loop(0, H, step=16)
        def _(j):
            rows0_v[r, pl.ds(j, 16)] = jnp.zeros((16,), jnp.float32)

    @pl.loop(0, NR, step=CHUNK)
    def _(r):
        pltpu.async_copy(rows0_v, acc_sh.at[pl.ds(s * NR + r, CHUNK)], semA)

    @pl.loop(0, NR, step=CHUNK)
    def _(r):
        pltpu.make_async_copy(rows0_v, acc_sh.at[pl.ds(0, CHUNK)],
                              semA).wait()

    pltpu.make_async_copy(src_hbm.at[w], sidx_v, semD0).wait()

    pltpu.async_copy(dst_hbm.at[w, 0], didx0_v, semD0)
    pltpu.async_copy(dst_hbm.at[w, 1], didx1_v, semD1)
    pltpu.async_copy(y_hbm.at[sidx_v.at[0]], rows0_v, semA)
    pltpu.async_copy(y_hbm.at[sidx_v.at[1]], rows1_v, semB)
    plsc.subcore_barrier()

    @pl.loop(0, NCH, step=2)
    def _(k):
        pltpu.make_async_copy(dst_hbm.at[w, 0], didx0_v, semD0).wait()
        pltpu.make_async_copy(y_hbm.at[sidx_v.at[0]], rows0_v, semA).wait()
        pltpu.sync_copy(rows0_v, acc_sh.at[didx0_v], add=True)

        @pl.when(k + 2 < NCH)
        def _():
            pltpu.async_copy(dst_hbm.at[w, k + 2], didx0_v, semD0)
            pltpu.async_copy(y_hbm.at[sidx_v.at[k + 2]], rows0_v, semA)

        pltpu.make_async_copy(dst_hbm.at[w, 0], didx1_v, semD1).wait()
        pltpu.make_async_copy(y_hbm.at[sidx_v.at[0]], rows1_v, semB).wait()
        pltpu.sync_copy(rows1_v, acc_sh.at[didx1_v], add=True)

        @pl.when(k + 3 < NCH)
        def _():
            pltpu.async_copy(dst_hbm.at[w, k + 3], didx1_v, semD1)
            pltpu.async_copy(y_hbm.at[sidx_v.at[k + 3]], rows1_v, semB)

    plsc.subcore_barrier()
    pltpu.sync_copy(acc_sh.at[pl.ds(s * NR, NR)],
                    out_hbm.at[c, pl.ds(s * NR, NR)])


def _sc_aggregate(src3, dst3, y):
    return pl.kernel(
        _agg_kernel,
        out_type=jax.ShapeDtypeStruct((NC, NP, H), jnp.float32),
        mesh=_mesh,
        compiler_params=_sc_params,
        scratch_types=[
            pltpu.VMEM((NCH, CHUNK), jnp.int32),
            pltpu.VMEM((CHUNK,), jnp.int32),
            pltpu.VMEM((CHUNK,), jnp.int32),
            pltpu.VMEM((CHUNK, H), jnp.float32),
            pltpu.VMEM((CHUNK, H), jnp.float32),
            pltpu.VMEM_SHARED((NP, H), jnp.float32),
            pltpu.SemaphoreType.DMA,
            pltpu.SemaphoreType.DMA,
            pltpu.SemaphoreType.DMA,
            pltpu.SemaphoreType.DMA,
        ],
    )(src3, dst3, y)


def _agg1_kernel(src_hbm, dst_hbm, y2_hbm, out_hbm, sidx_v, didx_v, y2_v,
                 vals0_v, vals1_v, zb_v, acc_sh, semA, semB):
    c = lax.axis_index("c")
    s = lax.axis_index("s")
    w = c * NS + s

    pltpu.async_copy(src_hbm.at[w], sidx_v, semA)
    pltpu.async_copy(dst_hbm.at[w], didx_v, semA)
    pltpu.async_copy(y2_hbm, y2_v, semA)

    @pl.loop(0, NR, step=16)
    def _(i):
        zb_v[pl.ds(i, 16)] = jnp.zeros((16,), jnp.float32)

    pltpu.sync_copy(zb_v, acc_sh.at[pl.ds(s * NR, NR)])
    pltpu.make_async_copy(src_hbm.at[w], sidx_v, semA).wait()
    pltpu.make_async_copy(dst_hbm.at[w], didx_v, semA).wait()
    pltpu.make_async_copy(y2_hbm, y2_v, semA).wait()
    plsc.subcore_barrier()

    def build(k, vals_v):
        @pl.loop(0, CHUNK, step=16)
        def _(j):
            idxv = sidx_v[k, pl.ds(j, 16)]
            vals_v[pl.ds(j, 16)] = plsc.load_gather(y2_v, [idxv])

    @pl.loop(0, NCH, step=2)
    def _(k):
        @pl.when(k >= 2)
        def _():
            pltpu.make_async_copy(vals0_v, acc_sh.at[didx_v.at[0]],
                                  semA).wait()

        build(k, vals0_v)
        pltpu.async_copy(vals0_v, acc_sh.at[didx_v.at[k]], semA, add=True)

        @pl.when(k >= 2)
        def _():
            pltpu.make_async_copy(vals1_v, acc_sh.at[didx_v.at[0]],
                                  semB).wait()

        build(k + 1, vals1_v)
        pltpu.async_copy(vals1_v, acc_sh.at[didx_v.at[k + 1]], semB, add=True)

    pltpu.make_async_copy(vals0_v, acc_sh.at[didx_v.at[0]], semA).wait()
    pltpu.make_async_copy(vals1_v, acc_sh.at[didx_v.at[0]], semB).wait()

    plsc.subcore_barrier()
    pltpu.sync_copy(acc_sh.at[pl.ds(s * NR, NR)],
                    out_hbm.at[c, pl.ds(s * NR, NR)])


def _sc_aggregate1(src3, dst3, y2):
    return pl.kernel(
        _agg1_kernel,
        out_type=jax.ShapeDtypeStruct((NC, NP), jnp.float32),
        mesh=_mesh,
        compiler_params=_sc_params,
        scratch_types=[
            pltpu.VMEM((NCH, CHUNK), jnp.int32),
            pltpu.VMEM((NCH, CHUNK), jnp.int32),
            pltpu.VMEM((NP,), jnp.float32),
            pltpu.VMEM((CHUNK,), jnp.float32),
            pltpu.VMEM((CHUNK,), jnp.float32),
            pltpu.VMEM((NR,), jnp.float32),
            pltpu.VMEM_SHARED((NP,), jnp.float32),
            pltpu.SemaphoreType.DMA,
            pltpu.SemaphoreType.DMA,
        ],
    )(src3, dst3, y2)


def _mm_scale_body(d_ref, x_ref, w_ref, y_ref, dis_ref):
    dis = lax.rsqrt(d_ref[0, :] + d_ref[1, :] + 1.0)
    t = jnp.dot(x_ref[...], w_ref[...], preferred_element_type=jnp.float32)
    y_ref[...] = t * dis[:, None]
    dis_ref[...] = dis[:, None]


def _tc_mm_scale(deg2, xp, W1):
    return pl.pallas_call(
        _mm_scale_body,
        grid=(NP // RB,),
        in_specs=[pl.BlockSpec((NC, RB), lambda i: (0, i)),
                  pl.BlockSpec((RB, F), lambda i: (i, 0)),
                  pl.BlockSpec((F, H), lambda i: (0, 0))],
        out_specs=[pl.BlockSpec((RB, H), lambda i: (i, 0)),
                   pl.BlockSpec((RB, 1), lambda i: (i, 0))],
        out_shape=[jax.ShapeDtypeStruct((NP, H), jnp.float32),
                   jax.ShapeDtypeStruct((NP, 1), jnp.float32)],
    )(deg2, xp, W1)


def _layer2_body(a_ref, y_ref, dis_ref, b1_ref, w2_ref, y2_ref):
    dis = dis_ref[...]
    h1 = dis * (a_ref[0] + a_ref[1] + y_ref[...]) + b1_ref[...]
    r = jnp.maximum(h1, 0.0)
    h2 = jnp.dot(r, w2_ref[...], preferred_element_type=jnp.float32)
    y2_ref[...] = h2 * dis


def _tc_layer2(aggp, y, dis, b1, W2):
    return pl.pallas_call(
        _layer2_body,
        grid=(NP // RB,),
        in_specs=[pl.BlockSpec((NC, RB, H), lambda i: (0, i, 0)),
                  pl.BlockSpec((RB, H), lambda i: (i, 0)),
                  pl.BlockSpec((RB, 1), lambda i: (i, 0)),
                  pl.BlockSpec((1, H), lambda i: (0, 0)),
                  pl.BlockSpec((H, 1), lambda i: (0, 0))],
        out_specs=pl.BlockSpec((RB, 1), lambda i: (i, 0)),
        out_shape=jax.ShapeDtypeStruct((NP, 1), jnp.float32),
    )(aggp, y, dis, b1, W2)


def _final_body(a_ref, y2_ref, dis_ref, b2_ref, o_ref):
    dis = dis_ref[...]
    agg = a_ref[0, :] + a_ref[1, :]
    o_ref[...] = dis * agg[:, None] + dis * y2_ref[...] + b2_ref[...]


def _tc_final(agg2p, y2, dis, b2):
    return pl.pallas_call(
        _final_body,
        grid=(NP // RB,),
        in_specs=[pl.BlockSpec((NC, RB), lambda i: (0, i)),
                  pl.BlockSpec((RB, 1), lambda i: (i, 0)),
                  pl.BlockSpec((RB, 1), lambda i: (i, 0)),
                  pl.BlockSpec((1, 1), lambda i: (0, 0))],
        out_specs=pl.BlockSpec((RB, 1), lambda i: (i, 0)),
        out_shape=jax.ShapeDtypeStruct((NP, 1), jnp.float32),
    )(agg2p, y2, dis, b2)


@jax.jit
def _run(x, edge_index, W1, b1, W2, b2):
    pad = jnp.arange(EP - E, dtype=jnp.int32) % (NP - N) + N
    srcp = jnp.concatenate([edge_index[0], pad])
    dstp = jnp.concatenate([edge_index[1], pad])
    src3 = srcp.reshape(NW, NCH, CHUNK)
    dst3 = dstp.reshape(NW, NCH, CHUNK)
    xp = jnp.pad(x, ((0, NP - N), (0, 0)))

    deg2 = _sc_degree(dst3)
    y, dis = _tc_mm_scale(deg2, xp, W1)
    aggp = _sc_aggregate(src3, dst3, y)
    y2 = _tc_layer2(aggp, y, dis, b1.reshape(1, H), W2)
    agg2p = _sc_aggregate1(src3, dst3, y2.reshape(NP))
    out = _tc_final(agg2p, y2, dis, b2.reshape(1, 1))
    return out[:N]


def kernel(x, edge_index, W1, b1, W2, b2):
    return _run(x, edge_index, W1, b1, W2, b2)


# speedup vs baseline: 1.0764x; 1.0764x over previous
import dataclasses

import jax
import jax.numpy as jnp
from jax import lax
from jax.experimental import pallas as pl
from jax.experimental.pallas import tpu as pltpu
from jax.experimental.pallas import tpu_sc as plsc

N = 10000
E = 320000
F = 128
H = 128
NP = 10240
NC = 2
NS = 16
NW = NC * NS
CHUNK = 128
NCH = 80
EPT = NCH * CHUNK
EP = NW * EPT
NR = NP // NS
RB = 1024

HALF = NP // NC
ACC2 = 5376
ZB2 = ACC2 // NS
MY = HALF // NS

_mesh = plsc.VectorSubcoreMesh(core_axis_name="c", subcore_axis_name="s")

_sc_params = pltpu.CompilerParams()
if "needs_layout_passes" in pltpu.CompilerParams.__dataclass_fields__:
    _sc_params = dataclasses.replace(_sc_params, needs_layout_passes=False)


def _deg_kernel(dst_hbm, out_hbm, didx_v, ones_v, zb_v, acc_sh, sem):
    c = lax.axis_index("c")
    s = lax.axis_index("s")
    w = c * NS + s

    pltpu.async_copy(dst_hbm.at[w], didx_v, sem)

    @pl.loop(0, CHUNK, step=16)
    def _(i):
        ones_v[pl.ds(i, 16)] = jnp.full((16,), 1.0, jnp.float32)

    @pl.loop(0, NR, step=16)
    def _(i):
        zb_v[pl.ds(i, 16)] = jnp.zeros((16,), jnp.float32)

    pltpu.sync_copy(zb_v, acc_sh.at[pl.ds(s * NR, NR)])
    pltpu.make_async_copy(dst_hbm.at[w], didx_v, sem).wait()
    plsc.subcore_barrier()

    @pl.loop(0, NCH)
    def _(k):
        pltpu.async_copy(ones_v, acc_sh.at[didx_v.at[k]], sem, add=True)

    @pl.loop(0, NCH)
    def _(k):
        pltpu.make_async_copy(ones_v, acc_sh.at[didx_v.at[0]], sem).wait()

    plsc.subcore_barrier()
    pltpu.sync_copy(acc_sh.at[pl.ds(s * NR, NR)],
                    out_hbm.at[c, pl.ds(s * NR, NR)])


def _sc_degree(dst3):
    return pl.kernel(
        _deg_kernel,
        out_type=jax.ShapeDtypeStruct((NC, NP), jnp.float32),
        mesh=_mesh,
        compiler_params=_sc_params,
        scratch_types=[
            pltpu.VMEM((NCH, CHUNK), jnp.int32),
            pltpu.VMEM((CHUNK,), jnp.float32),
            pltpu.VMEM((NR,), jnp.float32),
            pltpu.VMEM_SHARED((NP,), jnp.float32),
            pltpu.SemaphoreType.DMA,
        ],
    )(dst3)


QCH = 64
NQ4 = EPT // QCH


def _agg_kernel(src_hbm, dst_hbm, y_hbm, out_hbm, sidx_v,
                di0, di1, di2, di3, ro0, ro1, ro2, ro3, acc_sh,
                g0, g1, g2, g3):
    c = lax.axis_index("c")
    s = lax.axis_index("s")
    w = c * NS + s
    base = w * EPT
    dis = [di0, di1, di2, di3]
    ros = [ro0, ro1, ro2, ro3]
    gss = [g0, g1, g2, g3]

    pltpu.async_copy(src_hbm.at[pl.ds(base, EPT)], sidx_v, g0)

    @pl.loop(0, QCH)
    def _(r):
        @pl.loop(0, H, step=16)
        def _(j):
            ro0[r, pl.ds(j, 16)] = jnp.zeros((16,), jnp.float32)

    @pl.loop(0, NR, step=QCH)
    def _(r):
        pltpu.async_copy(ro0, acc_sh.at[pl.ds(s * NR + r, QCH)], g1)

    @pl.loop(0, NR, step=QCH)
    def _(r):
        pltpu.make_async_copy(ro0, acc_sh.at[pl.ds(0, QCH)], g1).wait()

    pltpu.make_async_copy(src_hbm.at[pl.ds(base, EPT)], sidx_v, g0).wait()

    def issue(q, b):
        pltpu.async_copy(dst_hbm.at[pl.ds(base + q * QCH, QCH)], dis[b],
                         gss[b])
        pltpu.async_copy(y_hbm.at[sidx_v.at[pl.ds(q * QCH, QCH)]],
                         ros[b], gss[b])

    for q in range(4):
        issue(q, q)
    plsc.subcore_barrier()

    @pl.loop(0, NQ4, step=4)
    def _(k):
        for b in range(4):
            q = k + b
            pltpu.make_async_copy(dst_hbm.at[pl.ds(base, QCH)], dis[b],
                                  gss[b]).wait()
            pltpu.make_async_copy(y_hbm.at[sidx_v.at[pl.ds(0, QCH)]],
                                  ros[b], gss[b]).wait()
            pltpu.sync_copy(ros[b], acc_sh.at[dis[b]], add=True)

            @pl.when(q + 4 < NQ4)
            def _():
                issue(q + 4, b)

    plsc.subcore_barrier()
    pltpu.sync_copy(acc_sh.at[pl.ds(s * NR, NR)],
                    out_hbm.at[c, pl.ds(s * NR, NR)])


def _sc_aggregate(srcp, dstp, y):
    return pl.kernel(
        _agg_kernel,
        out_type=jax.ShapeDtypeStruct((NC, NP, H), jnp.float32),
        mesh=_mesh,
        compiler_params=_sc_params,
        scratch_types=[
            pltpu.VMEM((EPT,), jnp.int32),
            pltpu.VMEM((QCH,), jnp.int32),
            pltpu.VMEM((QCH,), jnp.int32),
            pltpu.VMEM((QCH,), jnp.int32),
            pltpu.VMEM((QCH,), jnp.int32),
            pltpu.VMEM((QCH, H), jnp.float32),
            pltpu.VMEM((QCH, H), jnp.float32),
            pltpu.VMEM((QCH, H), jnp.float32),
            pltpu.VMEM((QCH, H), jnp.float32),
            pltpu.VMEM_SHARED((NP, H), jnp.float32),
            pltpu.SemaphoreType.DMA,
            pltpu.SemaphoreType.DMA,
            pltpu.SemaphoreType.DMA,
            pltpu.SemaphoreType.DMA,
        ],
    )(srcp, dstp, y)


def _agg1_kernel(src_hbm, dst_hbm, y2_hbm, out_hbm, sidx_v, didx_v, y2_v,
                 vals0_v, vals1_v, zb_v, acc_sh, semA, semB):
    c = lax.axis_index("c")
    s = lax.axis_index("s")
    w = c * NS + s

    pltpu.async_copy(src_hbm.at[w], sidx_v, semA)
    pltpu.async_copy(dst_hbm.at[w], didx_v, semA)
    pltpu.async_copy(y2_hbm, y2_v, semA)

    @pl.loop(0, NR, step=16)
    def _(i):
        zb_v[pl.ds(i, 16)] = jnp.zeros((16,), jnp.float32)

    pltpu.sync_copy(zb_v, acc_sh.at[pl.ds(s * NR, NR)])
    pltpu.make_async_copy(src_hbm.at[w], sidx_v, semA).wait()
    pltpu.make_async_copy(dst_hbm.at[w], didx_v, semA).wait()
    pltpu.make_async_copy(y2_hbm, y2_v, semA).wait()
    plsc.subcore_barrier()

    def build(k, vals_v):
        @pl.loop(0, CHUNK, step=16)
        def _(j):
            idxv = sidx_v[k, pl.ds(j, 16)]
            vals_v[pl.ds(j, 16)] = plsc.load_gather(y2_v, [idxv])

    @pl.loop(0, NCH, step=2)
    def _(k):
        @pl.when(k >= 2)
        def _():
            pltpu.make_async_copy(vals0_v, acc_sh.at[didx_v.at[0]],
                                  semA).wait()

        build(k, vals0_v)
        pltpu.async_copy(vals0_v, acc_sh.at[didx_v.at[k]], semA, add=True)

        @pl.when(k >= 2)
        def _():
            pltpu.make_async_copy(vals1_v, acc_sh.at[didx_v.at[0]],
                                  semB).wait()

        build(k + 1, vals1_v)
        pltpu.async_copy(vals1_v, acc_sh.at[didx_v.at[k + 1]], semB, add=True)

    pltpu.make_async_copy(vals0_v, acc_sh.at[didx_v.at[0]], semA).wait()
    pltpu.make_async_copy(vals1_v, acc_sh.at[didx_v.at[0]], semB).wait()

    plsc.subcore_barrier()
    pltpu.sync_copy(acc_sh.at[pl.ds(s * NR, NR)],
                    out_hbm.at[c, pl.ds(s * NR, NR)])


def _sc_aggregate1(src3, dst3, y2):
    return pl.kernel(
        _agg1_kernel,
        out_type=jax.ShapeDtypeStruct((NC, NP), jnp.float32),
        mesh=_mesh,
        compiler_params=_sc_params,
        scratch_types=[
            pltpu.VMEM((NCH, CHUNK), jnp.int32),
            pltpu.VMEM((NCH, CHUNK), jnp.int32),
            pltpu.VMEM((NP,), jnp.float32),
            pltpu.VMEM((CHUNK,), jnp.float32),
            pltpu.VMEM((CHUNK,), jnp.float32),
            pltpu.VMEM((NR,), jnp.float32),
            pltpu.VMEM_SHARED((NP,), jnp.float32),
            pltpu.SemaphoreType.DMA,
            pltpu.SemaphoreType.DMA,
        ],
    )(src3, dst3, y2)


def _mm_scale_body(d_ref, x_ref, w_ref, y_ref, dis_ref):
    dis = lax.rsqrt(d_ref[0, :] + d_ref[1, :] + 1.0)
    t = jnp.dot(x_ref[...], w_ref[...], preferred_element_type=jnp.float32)
    y_ref[...] = t * dis[:, None]
    dis_ref[...] = dis[:, None]


def _tc_mm_scale(deg2, xp, W1):
    return pl.pallas_call(
        _mm_scale_body,
        grid=(NP // RB,),
        in_specs=[pl.BlockSpec((NC, RB), lambda i: (0, i)),
                  pl.BlockSpec((RB, F), lambda i: (i, 0)),
                  pl.BlockSpec((F, H), lambda i: (0, 0))],
        out_specs=[pl.BlockSpec((RB, H), lambda i: (i, 0)),
                   pl.BlockSpec((RB, 1), lambda i: (i, 0))],
        out_shape=[jax.ShapeDtypeStruct((NP, H), jnp.float32),
                   jax.ShapeDtypeStruct((NP, 1), jnp.float32)],
    )(deg2, xp, W1)


def _layer2_body(a_ref, y_ref, dis_ref, b1_ref, w2_ref, y2_ref):
    dis = dis_ref[...]
    h1 = dis * (a_ref[0] + a_ref[1] + y_ref[...]) + b1_ref[...]
    r = jnp.maximum(h1, 0.0)
    h2 = jnp.dot(r, w2_ref[...], preferred_element_type=jnp.float32)
    y2_ref[...] = h2 * dis


def _tc_layer2(aggp, y, dis, b1, W2):
    return pl.pallas_call(
        _layer2_body,
        grid=(NP // RB,),
        in_specs=[pl.BlockSpec((NC, RB, H), lambda i: (0, i, 0)),
                  pl.BlockSpec((RB, H), lambda i: (i, 0)),
                  pl.BlockSpec((RB, 1), lambda i: (i, 0)),
                  pl.BlockSpec((1, H), lambda i: (0, 0)),
                  pl.BlockSpec((H, 1), lambda i: (0, 0))],
        out_specs=pl.BlockSpec((RB, 1), lambda i: (i, 0)),
        out_shape=jax.ShapeDtypeStruct((NP, 1), jnp.float32),
    )(aggp, y, dis, b1, W2)


def _final_body(a_ref, y2_ref, dis_ref, b2_ref, o_ref):
    dis = dis_ref[...]
    agg = a_ref[0, :] + a_ref[1, :]
    o_ref[...] = dis * agg[:, None] + dis * y2_ref[...] + b2_ref[...]


def _tc_final(agg2p, y2, dis, b2):
    return pl.pallas_call(
        _final_body,
        grid=(NP // RB,),
        in_specs=[pl.BlockSpec((NC, RB), lambda i: (0, i)),
                  pl.BlockSpec((RB, 1), lambda i: (i, 0)),
                  pl.BlockSpec((RB, 1), lambda i: (i, 0)),
                  pl.BlockSpec((1, 1), lambda i: (0, 0))],
        out_specs=pl.BlockSpec((RB, 1), lambda i: (i, 0)),
        out_shape=jax.ShapeDtypeStruct((NP, 1), jnp.float32),
    )(agg2p, y2, dis, b2)


@jax.jit
def _run(x, edge_index, W1, b1, W2, b2):
    pad = jnp.arange(EP - E, dtype=jnp.int32) % (NP - N) + N
    srcp = jnp.concatenate([edge_index[0], pad])
    dstp = jnp.concatenate([edge_index[1], pad])
    src3 = srcp.reshape(NW, NCH, CHUNK)
    dst3 = dstp.reshape(NW, NCH, CHUNK)
    xp = jnp.pad(x, ((0, NP - N), (0, 0)))

    deg2 = _sc_degree(dst3)
    y, dis = _tc_mm_scale(deg2, xp, W1)
    aggp = _sc_aggregate(srcp, dstp, y)
    y2 = _tc_layer2(aggp, y, dis, b1.reshape(1, H), W2)
    agg2p = _sc_aggregate1(src3, dst3, y2.reshape(NP))
    out = _tc_final(agg2p, y2, dis, b2.reshape(1, 1))
    return out[:N]


def kernel(x, edge_index, W1, b1, W2, b2):
    return _run(x, edge_index, W1, b1, W2, b2)
